# baseline (speedup 1.0000x reference)
import functools

import jax
import jax.numpy as jnp
from jax import lax
from jax.experimental import pallas as pl
from jax.experimental.pallas import tpu as pltpu
from jax.experimental.pallas import tpu_sc as plsc

B = 16384
L = 200
D = 64
LH = L // 2
NW = 32
BPW = B // NW
CH = 8
NBLK = BPW // CH
INV_L = 1.0 / L

_mesh = plsc.VectorSubcoreMesh(core_axis_name="c", subcore_axis_name="s")


@functools.partial(
    pl.kernel,
    mesh=_mesh,
    out_type=jax.ShapeDtypeStruct((B, D), jnp.float32),
    scratch_types=[
        pltpu.VMEM((2, 2 * CH, LH), jnp.int32),
        pltpu.VMEM((2, L, D), jnp.bfloat16),
        pltpu.VMEM((CH, D), jnp.float32),
        pltpu.VMEM((2 * D,), jnp.float32),
        pltpu.SemaphoreType.DMA,
        pltpu.SemaphoreType.DMA,
        pltpu.SemaphoreType.DMA,
        pltpu.SemaphoreType.DMA,
    ],
    compiler_params=pltpu.CompilerParams(
        use_tc_tiling_on_sc=False, needs_layout_passes=False),
)
def _encode(x2_hbm, table_hbm, out_hbm, idx_v, rows_v, out_v, scr_v,
            gsem0, gsem1, isem0, isem1):
    wid = lax.axis_index("s") * 2 + lax.axis_index("c")
    base = wid * BPW
    gsem = (gsem0, gsem1)
    isem = (isem0, isem1)

    lane = lax.iota(jnp.int32, 16)
    perm0 = (lane & 1) * 16 + (lane >> 1)

    def fire_gather(q, j, p):
        pltpu.async_copy(
            table_hbm.at[idx_v.at[q, 2 * j]], rows_v.at[p, pl.ds(0, LH)],
            gsem[p])
        pltpu.async_copy(
            table_hbm.at[idx_v.at[q, 2 * j + 1]], rows_v.at[p, pl.ds(LH, LH)],
            gsem[p])

    def wait_gather(q, j, p):
        pltpu.make_async_copy(
            table_hbm.at[idx_v.at[q, 2 * j]], rows_v.at[p, pl.ds(0, LH)],
            gsem[p]).wait()
        pltpu.make_async_copy(
            table_hbm.at[idx_v.at[q, 2 * j + 1]], rows_v.at[p, pl.ds(LH, LH)],
            gsem[p]).wait()

    def reduce_rows(p, j):
        def red_body(i, acc):
            accs = list(acc)
            for rr in range(8):
                r = i * 8 + rr
                for g in range(2):
                    ab = rows_v[p, r, pl.ds(32 * g, 32)]
                    a, b = plsc.unpack(ab, format=plsc.PackFormat.INTERLEAVED)
                    accs[2 * g] = accs[2 * g] + a
                    accs[2 * g + 1] = accs[2 * g + 1] + b
            return tuple(accs)

        zero = jnp.zeros((16,), jnp.float32)
        acc = lax.fori_loop(0, L // 8, red_body, (zero, zero, zero, zero))
        for g in range(2):
            scr_v[pl.ds(32 * g, 16)] = acc[2 * g] * INV_L
            scr_v[pl.ds(32 * g + 16, 16)] = acc[2 * g + 1] * INV_L
        for k in range(4):
            src = 32 * (k // 2) + 8 * (k % 2)
            out_v[j, pl.ds(16 * k, 16)] = plsc.load_gather(
                scr_v, [perm0 + src])

    def emit_block(blk, ip, fire_next, fire_idx):
        b0 = base + blk * CH
        for j in range(CH):
            p = j % 2
            if j < CH - 1:
                fire_gather(ip, j + 1, (j + 1) % 2)
            else:
                if fire_next:
                    pltpu.make_async_copy(
                        x2_hbm.at[pl.ds(2 * (b0 + CH), 2 * CH)],
                        idx_v.at[1 - ip], isem[1 - ip]).wait()
                    fire_gather(1 - ip, 0, 0)
                if fire_idx:
                    pltpu.async_copy(
                        x2_hbm.at[pl.ds(2 * (b0 + 2 * CH), 2 * CH)],
                        idx_v.at[ip], isem[ip])
            wait_gather(ip, j, p)
            reduce_rows(p, j)
        pltpu.sync_copy(out_v, out_hbm.at[pl.ds(b0, CH)])

    pltpu.sync_copy(x2_hbm.at[pl.ds(2 * base, 2 * CH)], idx_v.at[0])
    pltpu.async_copy(x2_hbm.at[pl.ds(2 * (base + CH), 2 * CH)],
                     idx_v.at[1], isem[1])
    fire_gather(0, 0, 0)

    def pair_body(k, _):
        emit_block(2 * k, 0, True, True)
        emit_block(2 * k + 1, 1, True, True)
        return 0

    lax.fori_loop(0, NBLK // 2 - 1, pair_body, 0)
    emit_block(NBLK - 2, 0, True, False)
    emit_block(NBLK - 1, 1, False, False)


def kernel(x, table):
    x2 = x.reshape(2 * B, LH)
    return _encode(x2, table.astype(jnp.bfloat16))


# speedup vs baseline: 1.2612x; 1.2612x over previous
import functools

import jax
import jax.numpy as jnp
from jax import lax
from jax.experimental import pallas as pl
from jax.experimental.pallas import tpu as pltpu
from jax.experimental.pallas import tpu_sc as plsc

B = 16384
L = 200
D = 64
LH1 = 128
LH2 = L - LH1
NW = 32
BPW = B // NW
CH = 8
NBLK = BPW // CH
NBUF = 4
INV_L = 1.0 / L

_mesh = plsc.VectorSubcoreMesh(core_axis_name="c", subcore_axis_name="s")


@functools.partial(
    pl.kernel,
    mesh=_mesh,
    out_type=jax.ShapeDtypeStruct((B, D), jnp.float32),
    scratch_types=[
        pltpu.VMEM((2, CH, L), jnp.int32),
        pltpu.VMEM((NBUF, L, D), jnp.float32),
        pltpu.VMEM((CH, D), jnp.float32),
        [pltpu.SemaphoreType.DMA] * NBUF,
        [pltpu.SemaphoreType.DMA] * 2,
    ],
    compiler_params=pltpu.CompilerParams(
        use_tc_tiling_on_sc=False, needs_layout_passes=False),
)
def _encode(x_hbm, table_hbm, out_hbm, idx_v, rows_v, out_v, gsem, isem):
    wid = lax.axis_index("s") * 2 + lax.axis_index("c")
    base = wid * BPW

    def fire_gather(q, j, p):
        pltpu.async_copy(
            table_hbm.at[idx_v.at[q, j, pl.ds(0, LH1)]],
            rows_v.at[p, pl.ds(0, LH1)], gsem[p])
        pltpu.async_copy(
            table_hbm.at[idx_v.at[q, j, pl.ds(LH1, LH2)]],
            rows_v.at[p, pl.ds(LH1, LH2)], gsem[p])

    def wait_gather(q, j, p):
        pltpu.make_async_copy(
            table_hbm.at[idx_v.at[q, j, pl.ds(0, LH1)]],
            rows_v.at[p, pl.ds(0, LH1)], gsem[p]).wait()
        pltpu.make_async_copy(
            table_hbm.at[idx_v.at[q, j, pl.ds(LH1, LH2)]],
            rows_v.at[p, pl.ds(LH1, LH2)], gsem[p]).wait()

    def reduce_rows(p, j):
        def red_body(i, acc):
            accs = list(acc)
            for rr in range(8):
                r = i * 8 + rr
                for c in range(4):
                    accs[c] = accs[c] + rows_v[p, r, pl.ds(c * 16, 16)]
            return tuple(accs)

        zero = jnp.zeros((16,), jnp.float32)
        acc = lax.fori_loop(0, L // 8, red_body, (zero, zero, zero, zero))
        for c in range(4):
            out_v[j, pl.ds(c * 16, 16)] = acc[c] * INV_L

    def emit_block(blk, ip, fire_next, fire_idx):
        b0 = base + blk * CH
        for j in range(CH):
            p = j % NBUF
            if j < CH - 2:
                fire_gather(ip, j + 2, (j + 2) % NBUF)
            elif j == CH - 2:
                if fire_next:
                    pltpu.make_async_copy(
                        x_hbm.at[pl.ds(b0 + CH, CH)],
                        idx_v.at[1 - ip], isem[1 - ip]).wait()
                    fire_gather(1 - ip, 0, 0)
            else:
                if fire_next:
                    fire_gather(1 - ip, 1, 1)
                if fire_idx:
                    pltpu.async_copy(
                        x_hbm.at[pl.ds(b0 + 2 * CH, CH)],
                        idx_v.at[ip], isem[ip])
            wait_gather(ip, j, p)
            reduce_rows(p, j)
        pltpu.sync_copy(out_v, out_hbm.at[pl.ds(b0, CH)])

    pltpu.sync_copy(x_hbm.at[pl.ds(base, CH)], idx_v.at[0])
    pltpu.async_copy(x_hbm.at[pl.ds(base + CH, CH)], idx_v.at[1], isem[1])
    fire_gather(0, 0, 0)
    fire_gather(0, 1, 1)

    def pair_body(k, _):
        emit_block(2 * k, 0, True, True)
        emit_block(2 * k + 1, 1, True, True)
        return 0

    lax.fori_loop(0, NBLK // 2 - 1, pair_body, 0)
    emit_block(NBLK - 2, 0, True, False)
    emit_block(NBLK - 1, 1, False, False)


def kernel(x, table):
    return _encode(x, table)
